# baseline (speedup 1.0000x reference)
import functools

import jax
import jax.numpy as jnp
from jax import lax
from jax.experimental import pallas as pl
from jax.experimental.pallas import tpu as pltpu
from jax.experimental.pallas import tpu_sc as plsc

_B, _S, _D = 4, 2048, 768
_N = _B * _S
_NW = 32
_SPW = _S // _NW
_WR = 8
_NWIN = _SPW // _WR
_WROWS = _B * _WR
_NQ = 4
_LANES = _D // 16


def _emb_body(idx_hbm, table_hbm, pos_hbm, out_hbm,
              idx_v, *bufs_and_sems):
    tok = bufs_and_sems[0:_NQ]
    posb = bufs_and_sems[_NQ:2 * _NQ]
    base = 2 * _NQ
    gsem = bufs_and_sems[base:base + _NQ]
    psem = bufs_and_sems[base + _NQ:base + 2 * _NQ]
    ssem = [bufs_and_sems[base + 2 * _NQ + s * _B:
                          base + 2 * _NQ + (s + 1) * _B]
            for s in range(_NQ)]
    isem = bufs_and_sems[base + 2 * _NQ + _NQ * _B]
    nc = 2
    wid = lax.axis_index("s") * nc + lax.axis_index("c")
    pos0 = wid * _SPW

    icp = [pltpu.async_copy(
               idx_hbm.at[pl.ds(b * _S + pos0 + w * _WR, _WR)],
               idx_v.at[pl.ds(w * _WROWS + b * _WR, _WR)], isem)
           for w in range(_NWIN) for b in range(_B)]

    def issue_gf(w, s):
        pltpu.async_copy(
            table_hbm.at[idx_v.at[pl.ds(w * _WROWS, _WROWS)]], tok[s],
            gsem[s])
        pltpu.async_copy(pos_hbm.at[pl.ds(pos0 + w * _WR, _WR)], posb[s],
                         psem[s])

    def wait_gf(w, s):
        pltpu.make_async_copy(
            table_hbm.at[idx_v.at[pl.ds(w * _WROWS, _WROWS)]], tok[s],
            gsem[s]).wait()
        pltpu.make_async_copy(pos_hbm.at[pl.ds(pos0 + w * _WR, _WR)],
                              posb[s], psem[s]).wait()

    def out_ref(w, b):
        return out_hbm.at[pl.ds(b * _S + pos0 + w * _WR, _WR)]

    def issue_stores(w, s):
        for b in range(_B):
            pltpu.async_copy(tok[s].at[pl.ds(b * _WR, _WR)], out_ref(w, b),
                             ssem[s][b])

    def wait_stores(w, s):
        for b in range(_B):
            pltpu.make_async_copy(tok[s].at[pl.ds(b * _WR, _WR)],
                                  out_ref(w, b), ssem[s][b]).wait()

    def add_rows(s):
        @plsc.parallel_loop(0, _WR)
        def row_body(r):
            for c in range(_LANES):
                sl = pl.ds(c * 16, 16)
                v = posb[s][r, sl]
                for b in range(_B):
                    plsc.addupdate(tok[s].at[b * _WR + r, sl], v)

    for c in icp:
        c.wait()
    issue_gf(0, 0)
    issue_gf(1, 1)

    for w in range(_NWIN):
        s = w % _NQ
        wait_gf(w, s)
        add_rows(s)
        issue_stores(w, s)
        if w + 2 < _NWIN:
            if w >= 2:
                wait_stores(w - 2, (w + 2) % _NQ)
            issue_gf(w + 2, (w + 2) % _NQ)

    for w in range(_NWIN - _NQ, _NWIN):
        wait_stores(w, w % _NQ)


@jax.jit
def _emb_lookup(idx_flat, token_table, pos_table):
    mesh = plsc.VectorSubcoreMesh(core_axis_name="c", subcore_axis_name="s")
    scratch = [pltpu.VMEM((_NWIN * _WROWS,), jnp.int32)]
    scratch += [pltpu.VMEM((_WROWS, _D), jnp.float32)
                for _ in range(_NQ)]
    scratch += [pltpu.VMEM((_WR, _D), jnp.float32)
                for _ in range(_NQ)]
    scratch += [pltpu.SemaphoreType.DMA
                for _ in range(2 * _NQ)]
    scratch += [pltpu.SemaphoreType.DMA
                for _ in range(_NQ * _B)]
    scratch += [pltpu.SemaphoreType.DMA]
    return pl.kernel(
        _emb_body,
        mesh=mesh,
        out_type=jax.ShapeDtypeStruct((_N, _D), jnp.float32),
        scratch_types=scratch,
    )(idx_flat, token_table, pos_table)


def kernel(embedding_idx, token_table, pos_table):
    b, s = embedding_idx.shape
    idx_flat = embedding_idx.reshape(b * s).astype(jnp.int32)
    out = _emb_lookup(idx_flat, token_table, pos_table)
    return out.reshape(b, s, token_table.shape[1])
